# baseline (speedup 1.0000x reference)
import jax
import jax.numpy as jnp
from jax.experimental import pallas as pl
from jax.experimental.pallas import tpu as pltpu

_N = 20000
_T = 1024
_NT = 20
_NPAD = _NT * _T
_IOU_T = 0.5


def _nms_tiles_kernel(x1r, y1r, x2r, y2r, ar, sr,
                      ox1, oy1, ox2, oy2, osc, marea):

    def tile_body(t, carry):
        ccx1 = x1r[t]
        ccy1 = y1r[t]
        ccx2 = x2r[t]
        ccy2 = y2r[t]
        ccar = ar[t]
        cx1 = ccx1.reshape(_T, 1)
        cy1 = ccy1.reshape(_T, 1)
        cx2 = ccx2.reshape(_T, 1)
        cy2 = ccy2.reshape(_T, 1)
        car = ccar.reshape(_T, 1)

        def cross_body(p, sup):
            px1 = ox1[p]
            py1 = oy1[p]
            px2 = ox2[p]
            py2 = oy2[p]
            par = marea[p]
            xx1 = jnp.maximum(cx1, px1)
            yy1 = jnp.maximum(cy1, py1)
            xx2 = jnp.minimum(cx2, px2)
            yy2 = jnp.minimum(cy2, py2)
            inter = jnp.maximum(xx2 - xx1, 0.0) * jnp.maximum(yy2 - yy1, 0.0)
            iou = inter / (car + par - inter + 1e-9)
            return jnp.maximum(sup, jnp.max(iou, axis=1, keepdims=True))

        sup0 = jax.lax.fori_loop(
            0, t, cross_body, jnp.zeros((_T, 1), jnp.float32))
        alive0_row = jnp.where(sup0 > _IOU_T, 0.0, 1.0)
        alive0_col = alive0_row.reshape(1, _T)

        xx1 = jnp.maximum(cx1, ccx1)
        yy1 = jnp.maximum(cy1, ccy1)
        xx2 = jnp.minimum(cx2, ccx2)
        yy2 = jnp.minimum(cy2, ccy2)
        inter = jnp.maximum(xx2 - xx1, 0.0) * jnp.maximum(yy2 - yy1, 0.0)
        iou = inter / (car + ccar - inter + 1e-9)
        row_i = jax.lax.broadcasted_iota(jnp.int32, (_T, _T), 0)
        col_i = jax.lax.broadcasted_iota(jnp.int32, (_T, _T), 1)
        e_mat = jnp.where((iou > _IOU_T) & (row_i < col_i), 1.0, 0.0)

        def jac_cond(state):
            return state[1] > 0

        def jac_body(state):
            a_col, _ = state
            a_row = a_col.reshape(_T, 1)
            sup = jnp.max(e_mat * a_row, axis=0, keepdims=True)
            a_new = jnp.where(sup > 0.0, 0.0, alive0_col)
            changed = jnp.sum(jnp.abs(a_new - a_col)).astype(jnp.float32)
            return a_new, changed

        a_final, _ = jax.lax.while_loop(
            jac_cond, jac_body, (alive0_col, jnp.float32(1.0)))

        marea[t] = ccar * a_final
        ox1[t] = ccx1 * a_final
        oy1[t] = ccy1 * a_final
        ox2[t] = ccx2 * a_final
        oy2[t] = ccy2 * a_final
        osc[t] = sr[t] * a_final
        return carry

    jax.lax.fori_loop(0, _NT, tile_body, 0)


def kernel(boxes, scores):
    order = jnp.argsort(-scores)
    bs = boxes[order]
    ss = scores[order]

    cx = bs[:, 0] * 1000.0
    cy = bs[:, 1] * 1000.0
    w = bs[:, 2] * 100.0 + 1.0
    h = bs[:, 3] * 100.0 + 1.0
    x1s = cx - 0.5 * w
    y1s = cy - 0.5 * h
    x2s = cx + 0.5 * w
    y2s = cy + 0.5 * h
    areas = jnp.maximum(x2s - x1s, 0.0) * jnp.maximum(y2s - y1s, 0.0)

    pad = _NPAD - _N

    def prep(v):
        return jnp.pad(v, (0, pad)).reshape(_NT, 1, _T)

    ins = [prep(v) for v in (x1s, y1s, x2s, y2s, areas, ss)]

    out_shape = [jax.ShapeDtypeStruct((_NT, 1, _T), jnp.float32)] * 5
    outs = pl.pallas_call(
        _nms_tiles_kernel,
        out_shape=out_shape,
        scratch_shapes=[pltpu.VMEM((_NT, 1, _T), jnp.float32)],
    )(*ins)

    cols = [o.reshape(_NPAD)[:_N] for o in outs]
    return jnp.stack(cols, axis=1)


# speedup vs baseline: 1.0254x; 1.0254x over previous
import jax
import jax.numpy as jnp
from jax.experimental import pallas as pl
from jax.experimental.pallas import tpu as pltpu

_N = 20000
_T = 1024
_NT = 20
_NPAD = _NT * _T
_IOU_T = 0.5


def _nms_tiles_kernel(x1r, y1r, x2r, y2r, ar, sr,
                      ox1, oy1, ox2, oy2, osc, marea):

    def tile_body(t, carry):
        ccx1 = x1r[t]
        ccy1 = y1r[t]
        ccx2 = x2r[t]
        ccy2 = y2r[t]
        ccar = ar[t]
        cx1 = ccx1.reshape(_T, 1)
        cy1 = ccy1.reshape(_T, 1)
        cx2 = ccx2.reshape(_T, 1)
        cy2 = ccy2.reshape(_T, 1)
        car = ccar.reshape(_T, 1)

        def cross_body(p, sup):
            px1 = ox1[p]
            py1 = oy1[p]
            px2 = ox2[p]
            py2 = oy2[p]
            par = marea[p]
            xx1 = jnp.maximum(cx1, px1)
            yy1 = jnp.maximum(cy1, py1)
            xx2 = jnp.minimum(cx2, px2)
            yy2 = jnp.minimum(cy2, py2)
            inter = jnp.maximum(xx2 - xx1, 0.0) * jnp.maximum(yy2 - yy1, 0.0)
            iou = inter / (car + par - inter + 1e-9)
            return jnp.maximum(sup, jnp.max(iou, axis=1, keepdims=True))

        sup0 = jax.lax.fori_loop(
            0, t, cross_body, jnp.zeros((_T, 1), jnp.float32))
        alive0_row = jnp.where(sup0 > _IOU_T, 0.0, 1.0)
        alive0_col = alive0_row.reshape(1, _T)

        xx1 = jnp.maximum(cx1, ccx1)
        yy1 = jnp.maximum(cy1, ccy1)
        xx2 = jnp.minimum(cx2, ccx2)
        yy2 = jnp.minimum(cy2, ccy2)
        inter = jnp.maximum(xx2 - xx1, 0.0) * jnp.maximum(yy2 - yy1, 0.0)
        iou = inter / (car + ccar - inter + 1e-9)
        row_i = jax.lax.broadcasted_iota(jnp.int32, (_T, _T), 0)
        col_i = jax.lax.broadcasted_iota(jnp.int32, (_T, _T), 1)
        e_mat = jnp.where((iou > _IOU_T) & (row_i < col_i), 1.0, 0.0)

        def jac_cond(state):
            return state[1] > 0

        def jac_body(state):
            a_col, _ = state
            a_row = a_col.reshape(_T, 1)
            sup = jnp.max(e_mat * a_row, axis=0, keepdims=True)
            a_new = jnp.where(sup > 0.0, 0.0, alive0_col)
            changed = jnp.sum(jnp.abs(a_new - a_col)).astype(jnp.float32)
            return a_new, changed

        a_final, _ = jax.lax.while_loop(
            jac_cond, jac_body, (alive0_col, jnp.float32(1.0)))

        marea[t] = ccar * a_final
        ox1[t] = ccx1 * a_final
        oy1[t] = ccy1 * a_final
        ox2[t] = ccx2 * a_final
        oy2[t] = ccy2 * a_final
        osc[t] = sr[t] * a_final
        return carry

    jax.lax.fori_loop(0, _NT, tile_body, 0)


def kernel(boxes, scores):
    cx = boxes[:, 0] * 1000.0
    cy = boxes[:, 1] * 1000.0
    w = boxes[:, 2] * 100.0 + 1.0
    h = boxes[:, 3] * 100.0 + 1.0
    x1 = cx - 0.5 * w
    y1 = cy - 0.5 * h
    x2 = cx + 0.5 * w
    y2 = cy + 0.5 * h

    order = jnp.argsort(-scores)
    x1s = x1[order]
    y1s = y1[order]
    x2s = x2[order]
    y2s = y2[order]
    ss = scores[order]
    areas = jnp.maximum(x2s - x1s, 0.0) * jnp.maximum(y2s - y1s, 0.0)

    pad = _NPAD - _N

    def prep(v):
        return jnp.pad(v, (0, pad)).reshape(_NT, 1, _T)

    ins = [prep(v) for v in (x1s, y1s, x2s, y2s, areas, ss)]

    out_shape = [jax.ShapeDtypeStruct((_NT, 1, _T), jnp.float32)] * 5
    outs = pl.pallas_call(
        _nms_tiles_kernel,
        out_shape=out_shape,
        scratch_shapes=[pltpu.VMEM((_NT, 1, _T), jnp.float32)],
    )(*ins)

    cols = [o.reshape(_NPAD)[:_N] for o in outs]
    return jnp.stack(cols, axis=1)
